# baseline (speedup 1.0000x reference)
import functools

import jax
import jax.numpy as jnp
from jax import lax
from jax.experimental import pallas as pl
from jax.experimental.pallas import tpu as pltpu
from jax.experimental.pallas import tpu_sc as plsc

NB = 2
E = 8
D = 768
F = 1024
T = 2048
TB = 256
BM = 128
NBLK = T // BM + E
PAD = NBLK * BM
PB = 256

_SC_WORKERS = 32
_SC_CHUNK = 64


def _layernorm(xb, g, b, eps=1e-5):
    m = jnp.mean(xb, axis=-1, keepdims=True)
    v = jnp.mean((xb - m) ** 2, axis=-1, keepdims=True)
    return (xb - m) / jnp.sqrt(v + eps) * g + b


def _routing_body(x_ref, g_ref, b_ref, gw_ref, gb_ref,
                  gate_ref, idx_ref, cumtok_ref, counts_ref, carry_ref):
    i = pl.program_id(1)

    @pl.when(i == 0)
    def _():
        carry_ref[...] = jnp.zeros((1, E), jnp.float32)

    xb = x_ref[...]
    h = _layernorm(xb, g_ref[0, 0, :], b_ref[0, 0, :])
    logits = jnp.dot(h, gw_ref[0], preferred_element_type=jnp.float32)
    logits = logits + gb_ref[0, 0, :]
    lmax = jnp.max(logits, axis=-1, keepdims=True)
    p = jnp.exp(logits - lmax)
    probs = p / jnp.sum(p, axis=-1, keepdims=True)
    gate = jnp.max(probs, axis=-1)
    iota_e = lax.broadcasted_iota(jnp.int32, (TB, E), 1)
    is_max = probs == gate[:, None]
    idxv = jnp.min(jnp.where(is_max, iota_e, E), axis=-1)
    onehot = (iota_e == idxv[:, None]).astype(jnp.float32)

    r_i = lax.broadcasted_iota(jnp.int32, (TB, TB), 0)
    c_i = lax.broadcasted_iota(jnp.int32, (TB, TB), 1)
    lt = (c_i < r_i).astype(jnp.float32)
    cumw = jnp.dot(lt, onehot, preferred_element_type=jnp.float32)
    carry = carry_ref[...]
    cum_excl = cumw + carry
    cumtok = jnp.sum(onehot * cum_excl, axis=-1)

    gate_ref[0, 0, :] = gate
    idx_ref[0, 0, :] = idxv
    cumtok_ref[0, 0, :] = cumtok.astype(jnp.int32)
    new_carry = carry + jnp.sum(onehot, axis=0, keepdims=True)
    carry_ref[...] = new_carry
    counts_ref[0, 0, :] = new_carry[0, :].astype(jnp.int32)


def _routing(x2d, ln_g, ln_b, gateW, gateb):
    return pl.pallas_call(
        _routing_body,
        grid=(NB, T // TB),
        in_specs=[
            pl.BlockSpec((TB, D), lambda b, i: (i, 0)),
            pl.BlockSpec((1, 1, D), lambda b, i: (b, 0, 0)),
            pl.BlockSpec((1, 1, D), lambda b, i: (b, 0, 0)),
            pl.BlockSpec((1, D, E), lambda b, i: (b, 0, 0)),
            pl.BlockSpec((1, 1, E), lambda b, i: (b, 0, 0)),
        ],
        out_specs=[
            pl.BlockSpec((1, 1, TB), lambda b, i: (b, 0, i)),
            pl.BlockSpec((1, 1, TB), lambda b, i: (b, 0, i)),
            pl.BlockSpec((1, 1, TB), lambda b, i: (b, 0, i)),
            pl.BlockSpec((1, 1, E), lambda b, i: (b, 0, 0)),
        ],
        out_shape=[
            jax.ShapeDtypeStruct((NB, 1, T), jnp.float32),
            jax.ShapeDtypeStruct((NB, 1, T), jnp.int32),
            jax.ShapeDtypeStruct((NB, 1, T), jnp.int32),
            jax.ShapeDtypeStruct((NB, 1, E), jnp.int32),
        ],
        scratch_shapes=[pltpu.VMEM((1, E), jnp.float32)],
    )(x2d, ln_g.reshape(NB, 1, D), ln_b.reshape(NB, 1, D), gateW,
      gateb.reshape(NB, 1, E))


def _finalize_body(counts_ref, idx_ref, cumtok_ref,
                   perm_ref, dest_ref, be_ref):
    b = pl.program_id(0)
    j = pl.program_id(1)

    counts = counts_ref[0, 0, :].astype(jnp.int32)
    pc = ((counts + (BM - 1)) // BM) * BM
    e_r = lax.broadcasted_iota(jnp.int32, (E, E), 0)
    e_c = lax.broadcasted_iota(jnp.int32, (E, E), 1)
    lt8 = (e_c < e_r).astype(jnp.float32)
    off = jnp.sum(lt8 * pc.astype(jnp.float32)[None, :], axis=-1)

    idxv = idx_ref[0, 0, :]
    cumtok = cumtok_ref[0, 0, :]
    iota_te = lax.broadcasted_iota(jnp.int32, (T, E), 1)
    oh = (iota_te == idxv[:, None]).astype(jnp.float32)
    off_tok = jnp.sum(oh * off[None, :], axis=-1)
    dest_local = cumtok + off_tok.astype(jnp.int32)

    @pl.when(j == 0)
    def _():
        dest_ref[0, 0, :] = dest_local + b * PAD
        jb = lax.broadcasted_iota(jnp.int32, (NBLK, E), 0) * BM
        off_be = jnp.broadcast_to(off[None, :], (NBLK, E))
        be = jnp.sum((off_be <= jb.astype(jnp.float32)).astype(jnp.int32),
                     axis=-1) - 1
        be_ref[0, 0, :] = jnp.clip(be, 0, E - 1)

    s_col = lax.broadcasted_iota(jnp.int32, (PB, T), 0) + j * PB
    d_row = jnp.broadcast_to(dest_local[None, :], (PB, T))
    ind = (s_col == d_row).astype(jnp.float32)
    t_row = lax.broadcasted_iota(jnp.int32, (PB, T), 1).astype(jnp.float32)
    perm_blk = jnp.sum(ind * t_row, axis=-1)
    perm_ref[0, 0, :] = perm_blk.astype(jnp.int32)


def _finalize(counts, idxs, cumtok):
    return pl.pallas_call(
        _finalize_body,
        grid=(NB, PAD // PB),
        in_specs=[
            pl.BlockSpec((1, 1, E), lambda b, j: (b, 0, 0)),
            pl.BlockSpec((1, 1, T), lambda b, j: (b, 0, 0)),
            pl.BlockSpec((1, 1, T), lambda b, j: (b, 0, 0)),
        ],
        out_specs=[
            pl.BlockSpec((1, 1, PB), lambda b, j: (b, 0, j)),
            pl.BlockSpec((1, 1, T), lambda b, j: (b, 0, 0)),
            pl.BlockSpec((1, 1, NBLK), lambda b, j: (b, 0, 0)),
        ],
        out_shape=[
            jax.ShapeDtypeStruct((NB, 1, PAD), jnp.int32),
            jax.ShapeDtypeStruct((NB, 1, T), jnp.int32),
            jax.ShapeDtypeStruct((NB, 1, NBLK), jnp.int32),
        ],
    )(counts, idxs, cumtok)


def _sc_gather(table, idx):
    n = idx.shape[0]
    d = table.shape[1]
    per_w = n // _SC_WORKERS
    n_chunks = per_w // _SC_CHUNK
    mesh = plsc.VectorSubcoreMesh(core_axis_name="c", subcore_axis_name="s")

    @functools.partial(
        pl.kernel,
        mesh=mesh,
        out_type=jax.ShapeDtypeStruct((n, d), jnp.float32),
        scratch_types=[
            pltpu.VMEM((_SC_CHUNK,), jnp.int32),
            pltpu.VMEM((_SC_CHUNK, d), jnp.float32),
            pltpu.SemaphoreType.DMA,
        ],
    )
    def k(table_hbm, idx_hbm, out_hbm, idx_v, rows_v, sem):
        wid = lax.axis_index("s") * 2 + lax.axis_index("c")
        base = wid * per_w
        for j in range(n_chunks):
            off = base + j * _SC_CHUNK
            pltpu.sync_copy(idx_hbm.at[pl.ds(off, _SC_CHUNK)], idx_v)
            pltpu.async_copy(table_hbm.at[idx_v], rows_v, sem).wait()
            pltpu.sync_copy(rows_v, out_hbm.at[pl.ds(off, _SC_CHUNK)])

    return k(table, idx)


def _ffn_body(be_ref, x_ref, g_ref, b_ref, w1_ref, b1_ref, w2_ref, b2_ref,
              out_ref):
    xb = x_ref[0]
    h = _layernorm(xb, g_ref[0, 0, :], b_ref[0, 0, :])
    h1 = jnp.dot(h, w1_ref[0, 0], preferred_element_type=jnp.float32)
    h1 = jax.nn.gelu(h1 + b1_ref[0, 0, 0])
    h2 = jnp.dot(h1, w2_ref[0, 0], preferred_element_type=jnp.float32)
    out_ref[0] = h2 + b2_ref[0, 0, 0]


def _ffn(be, sorted_x, ln_g, ln_b, W1, b1, W2, b2):
    grid_spec = pltpu.PrefetchScalarGridSpec(
        num_scalar_prefetch=1,
        grid=(NB, NBLK),
        in_specs=[
            pl.BlockSpec((1, BM, D), lambda b, j, s: (b, j, 0)),
            pl.BlockSpec((1, 1, D), lambda b, j, s: (b, 0, 0)),
            pl.BlockSpec((1, 1, D), lambda b, j, s: (b, 0, 0)),
            pl.BlockSpec((1, 1, D, F), lambda b, j, s: (b, s[b, j], 0, 0)),
            pl.BlockSpec((1, 1, 1, F), lambda b, j, s: (b, s[b, j], 0, 0)),
            pl.BlockSpec((1, 1, F, D), lambda b, j, s: (b, s[b, j], 0, 0)),
            pl.BlockSpec((1, 1, 1, D), lambda b, j, s: (b, s[b, j], 0, 0)),
        ],
        out_specs=pl.BlockSpec((1, BM, D), lambda b, j, s: (b, j, 0)),
    )
    return pl.pallas_call(
        _ffn_body,
        grid_spec=grid_spec,
        out_shape=jax.ShapeDtypeStruct((NB, PAD, D), jnp.float32),
    )(be, sorted_x, ln_g.reshape(NB, 1, D), ln_b.reshape(NB, 1, D),
      W1, b1.reshape(NB, E, 1, F), W2, b2.reshape(NB, E, 1, D))


def _combine_body(x_ref, moe_ref, gate_ref, emb_ref, embW_ref, embb_ref,
                  sg_ref, sb_ref, outW_ref, outb_ref, out_ref):
    g0 = gate_ref[0, 0, :]
    g1 = gate_ref[1, 0, :]
    u = (moe_ref[0] * g0[:, None] + moe_ref[1] * g1[:, None]) * 0.5
    e = jax.nn.silu(emb_ref[...])
    e = jnp.dot(e, embW_ref[...], preferred_element_type=jnp.float32)
    e = e + embb_ref[...]
    scale = e[:, :D]
    shift = e[:, D:]
    hh = _layernorm(u, sg_ref[0, :], sb_ref[0, :]) * (1.0 + scale) + shift
    hh = jax.nn.silu(hh)
    o = jnp.dot(hh, outW_ref[...], preferred_element_type=jnp.float32)
    out_ref[...] = x_ref[...] + o + outb_ref[...]


def _combine(x2d, moe, gate, emb, embW, embb, sn_g, sn_b, outW, outb):
    tdim = embW.shape[0]
    return pl.pallas_call(
        _combine_body,
        grid=(T // TB,),
        in_specs=[
            pl.BlockSpec((TB, D), lambda i: (i, 0)),
            pl.BlockSpec((NB, TB, D), lambda i: (0, i, 0)),
            pl.BlockSpec((NB, 1, TB), lambda i: (0, 0, i)),
            pl.BlockSpec((1, tdim), lambda i: (0, 0)),
            pl.BlockSpec((tdim, 2 * D), lambda i: (0, 0)),
            pl.BlockSpec((1, 2 * D), lambda i: (0, 0)),
            pl.BlockSpec((1, D), lambda i: (0, 0)),
            pl.BlockSpec((1, D), lambda i: (0, 0)),
            pl.BlockSpec((D, D), lambda i: (0, 0)),
            pl.BlockSpec((1, D), lambda i: (0, 0)),
        ],
        out_specs=pl.BlockSpec((TB, D), lambda i: (i, 0)),
        out_shape=jax.ShapeDtypeStruct((T, D), jnp.float32),
    )(x2d, moe, gate, emb, embW, embb.reshape(1, 2 * D),
      sn_g.reshape(1, D), sn_b.reshape(1, D), outW, outb.reshape(1, D))


def kernel(x, emb, ln_g, ln_b, gateW, gateb, W1, b1, W2, b2,
           embW, embb, sn_g, sn_b, outW, outb):
    x2d = x.reshape(T, D)

    gate, idxs, cumtok, counts = _routing(x2d, ln_g, ln_b, gateW, gateb)
    perm, dest, be = _finalize(counts, idxs, cumtok)

    sorted_x = _sc_gather(x2d, perm.reshape(NB * PAD))
    sorted_out = _ffn(be.reshape(NB, NBLK), sorted_x.reshape(NB, PAD, D),
                      ln_g, ln_b, W1, b1, W2, b2)
    moe = _sc_gather(sorted_out.reshape(NB * PAD, D), dest.reshape(NB * T))

    out = _combine(x2d, moe.reshape(NB, T, D), gate, emb, embW, embb,
                   sn_g, sn_b, outW, outb)
    return out.reshape(1, T, D)


# speedup vs baseline: 1.1192x; 1.1192x over previous
import functools

import jax
import jax.numpy as jnp
from jax import lax
from jax.experimental import pallas as pl
from jax.experimental.pallas import tpu as pltpu
from jax.experimental.pallas import tpu_sc as plsc

NB = 2
E = 8
D = 768
F = 1024
T = 2048
TB = 256
BM = 128
NBLK = T // BM + E
PAD = NBLK * BM
PB = 256

_SC_WORKERS = 32
_SC_CHUNK = 64


def _layernorm(xb, g, b, eps=1e-5):
    m = jnp.mean(xb, axis=-1, keepdims=True)
    v = jnp.mean((xb - m) ** 2, axis=-1, keepdims=True)
    return (xb - m) / jnp.sqrt(v + eps) * g + b


def _routing_body(x_ref, g_ref, b_ref, gw_ref, gb_ref,
                  gate_ref, idx_ref, cumtok_ref, counts_ref, carry_ref):
    i = pl.program_id(1)

    @pl.when(i == 0)
    def _():
        carry_ref[...] = jnp.zeros((1, E), jnp.float32)

    xb = x_ref[...]
    h = _layernorm(xb, g_ref[0, 0, :], b_ref[0, 0, :])
    logits = jnp.dot(h, gw_ref[0], preferred_element_type=jnp.float32)
    logits = logits + gb_ref[0, 0, :]
    lmax = jnp.max(logits, axis=-1, keepdims=True)
    p = jnp.exp(logits - lmax)
    probs = p / jnp.sum(p, axis=-1, keepdims=True)
    gate = jnp.max(probs, axis=-1)
    iota_e = lax.broadcasted_iota(jnp.int32, (TB, E), 1)
    is_max = probs == gate[:, None]
    idxv = jnp.min(jnp.where(is_max, iota_e, E), axis=-1)
    onehot = (iota_e == idxv[:, None]).astype(jnp.float32)

    r_i = lax.broadcasted_iota(jnp.int32, (TB, TB), 0)
    c_i = lax.broadcasted_iota(jnp.int32, (TB, TB), 1)
    lt = (c_i < r_i).astype(jnp.float32)
    cumw = jnp.dot(lt, onehot, preferred_element_type=jnp.float32)
    carry = carry_ref[...]
    cum_excl = cumw + carry
    cumtok = jnp.sum(onehot * cum_excl, axis=-1)

    gate_ref[0, 0, :] = gate
    idx_ref[0, 0, :] = idxv
    cumtok_ref[0, 0, :] = cumtok.astype(jnp.int32)
    new_carry = carry + jnp.sum(onehot, axis=0, keepdims=True)
    carry_ref[...] = new_carry
    counts_ref[0, 0, :] = new_carry[0, :].astype(jnp.int32)


def _routing(x2d, ln_g, ln_b, gateW, gateb):
    return pl.pallas_call(
        _routing_body,
        grid=(NB, T // TB),
        in_specs=[
            pl.BlockSpec((TB, D), lambda b, i: (i, 0)),
            pl.BlockSpec((1, 1, D), lambda b, i: (b, 0, 0)),
            pl.BlockSpec((1, 1, D), lambda b, i: (b, 0, 0)),
            pl.BlockSpec((1, D, E), lambda b, i: (b, 0, 0)),
            pl.BlockSpec((1, 1, E), lambda b, i: (b, 0, 0)),
        ],
        out_specs=[
            pl.BlockSpec((1, 1, TB), lambda b, i: (b, 0, i)),
            pl.BlockSpec((1, 1, TB), lambda b, i: (b, 0, i)),
            pl.BlockSpec((1, 1, TB), lambda b, i: (b, 0, i)),
            pl.BlockSpec((1, 1, E), lambda b, i: (b, 0, 0)),
        ],
        out_shape=[
            jax.ShapeDtypeStruct((NB, 1, T), jnp.float32),
            jax.ShapeDtypeStruct((NB, 1, T), jnp.int32),
            jax.ShapeDtypeStruct((NB, 1, T), jnp.int32),
            jax.ShapeDtypeStruct((NB, 1, E), jnp.int32),
        ],
        scratch_shapes=[pltpu.VMEM((1, E), jnp.float32)],
    )(x2d, ln_g.reshape(NB, 1, D), ln_b.reshape(NB, 1, D), gateW,
      gateb.reshape(NB, 1, E))


def _finalize_body(counts_ref, idx_ref, cumtok_ref,
                   perm_ref, dest_ref, be_ref):
    b = pl.program_id(0)
    j = pl.program_id(1)

    counts = counts_ref[0, 0, :].astype(jnp.int32)
    pc = ((counts + (BM - 1)) // BM) * BM
    e_r = lax.broadcasted_iota(jnp.int32, (E, E), 0)
    e_c = lax.broadcasted_iota(jnp.int32, (E, E), 1)
    lt8 = (e_c < e_r).astype(jnp.float32)
    off = jnp.sum(lt8 * pc.astype(jnp.float32)[None, :], axis=-1)

    idxv = idx_ref[0, 0, :]
    cumtok = cumtok_ref[0, 0, :]
    iota_te = lax.broadcasted_iota(jnp.int32, (T, E), 1)
    oh = (iota_te == idxv[:, None]).astype(jnp.float32)
    off_tok = jnp.sum(oh * off[None, :], axis=-1)
    dest_local = cumtok + off_tok.astype(jnp.int32)

    @pl.when(j == 0)
    def _():
        dest_ref[0, 0, :] = dest_local
        jb = lax.broadcasted_iota(jnp.int32, (NBLK, E), 0) * BM
        off_be = jnp.broadcast_to(off[None, :], (NBLK, E))
        be = jnp.sum((off_be <= jb.astype(jnp.float32)).astype(jnp.int32),
                     axis=-1) - 1
        be_ref[0, 0, :] = jnp.clip(be, 0, E - 1)

    s_col = lax.broadcasted_iota(jnp.int32, (PB, T), 0) + j * PB
    d_row = jnp.broadcast_to(dest_local[None, :], (PB, T))
    ind = (s_col == d_row).astype(jnp.float32)
    t_row = lax.broadcasted_iota(jnp.int32, (PB, T), 1).astype(jnp.float32)
    perm_blk = jnp.sum(ind * t_row, axis=-1)
    matched = jnp.sum(ind, axis=-1)
    slot_ids = lax.broadcasted_iota(jnp.int32, (PB, 1), 0)[:, 0] + j * PB
    fallback = (slot_ids % T).astype(jnp.float32)
    perm_blk = perm_blk + (1.0 - matched) * fallback
    perm_ref[0, 0, :] = perm_blk.astype(jnp.int32)


def _finalize(counts, idxs, cumtok):
    return pl.pallas_call(
        _finalize_body,
        grid=(NB, PAD // PB),
        in_specs=[
            pl.BlockSpec((1, 1, E), lambda b, j: (b, 0, 0)),
            pl.BlockSpec((1, 1, T), lambda b, j: (b, 0, 0)),
            pl.BlockSpec((1, 1, T), lambda b, j: (b, 0, 0)),
        ],
        out_specs=[
            pl.BlockSpec((1, 1, PB), lambda b, j: (b, 0, j)),
            pl.BlockSpec((1, 1, T), lambda b, j: (b, 0, 0)),
            pl.BlockSpec((1, 1, NBLK), lambda b, j: (b, 0, 0)),
        ],
        out_shape=[
            jax.ShapeDtypeStruct((NB, 1, PAD), jnp.int32),
            jax.ShapeDtypeStruct((NB, 1, T), jnp.int32),
            jax.ShapeDtypeStruct((NB, 1, NBLK), jnp.int32),
        ],
    )(counts, idxs, cumtok)


def _sc_gather(table, idx):
    n = idx.shape[0]
    d = table.shape[1]
    per_w = n // _SC_WORKERS
    mesh = plsc.VectorSubcoreMesh(core_axis_name="c", subcore_axis_name="s")

    @functools.partial(
        pl.kernel,
        mesh=mesh,
        out_type=jax.ShapeDtypeStruct((n, d), jnp.float32),
        scratch_types=[
            pltpu.VMEM((per_w,), jnp.int32),
            pltpu.VMEM((per_w, d), jnp.float32),
            pltpu.SemaphoreType.DMA,
        ],
    )
    def k(table_hbm, idx_hbm, out_hbm, idx_v, rows_v, sem):
        wid = lax.axis_index("s") * 2 + lax.axis_index("c")
        base = wid * per_w
        pltpu.sync_copy(idx_hbm.at[pl.ds(base, per_w)], idx_v)
        pltpu.async_copy(table_hbm.at[idx_v], rows_v, sem).wait()
        pltpu.sync_copy(rows_v, out_hbm.at[pl.ds(base, per_w)])

    return k(table, idx)


def _ffn_body(be_ref, x_ref, g_ref, b_ref, w1_ref, b1_ref, w2_ref, b2_ref,
              out_ref):
    xb = x_ref[0]
    h = _layernorm(xb, g_ref[0, 0, :], b_ref[0, 0, :])
    h1 = jnp.dot(h, w1_ref[0, 0], preferred_element_type=jnp.float32)
    h1 = jax.nn.gelu(h1 + b1_ref[0, 0, 0])
    h2 = jnp.dot(h1, w2_ref[0, 0], preferred_element_type=jnp.float32)
    out_ref[0] = h2 + b2_ref[0, 0, 0]


def _ffn(be, sorted_x, ln_g, ln_b, W1, b1, W2, b2):
    grid_spec = pltpu.PrefetchScalarGridSpec(
        num_scalar_prefetch=1,
        grid=(NBLK,),
        in_specs=[
            pl.BlockSpec((1, BM, D), lambda j, s: (0, j, 0)),
            pl.BlockSpec((1, 1, D), lambda j, s: (0, 0, 0)),
            pl.BlockSpec((1, 1, D), lambda j, s: (0, 0, 0)),
            pl.BlockSpec((1, 1, D, F), lambda j, s: (0, s[j], 0, 0)),
            pl.BlockSpec((1, 1, 1, F), lambda j, s: (0, s[j], 0, 0)),
            pl.BlockSpec((1, 1, F, D), lambda j, s: (0, s[j], 0, 0)),
            pl.BlockSpec((1, 1, 1, D), lambda j, s: (0, s[j], 0, 0)),
        ],
        out_specs=pl.BlockSpec((1, BM, D), lambda j, s: (0, j, 0)),
    )
    return pl.pallas_call(
        _ffn_body,
        grid_spec=grid_spec,
        out_shape=jax.ShapeDtypeStruct((1, PAD, D), jnp.float32),
    )(be, sorted_x.reshape(1, PAD, D), ln_g.reshape(1, 1, D),
      ln_b.reshape(1, 1, D), W1.reshape(1, E, D, F), b1.reshape(1, E, 1, F),
      W2.reshape(1, E, F, D), b2.reshape(1, E, 1, D))


def _combine_body(x_ref, moe0_ref, moe1_ref, gate_ref, emb_ref, embW_ref,
                  embb_ref, sg_ref, sb_ref, outW_ref, outb_ref, out_ref):
    g0 = gate_ref[0, 0, :]
    g1 = gate_ref[1, 0, :]
    u = (moe0_ref[...] * g0[:, None] + moe1_ref[...] * g1[:, None]) * 0.5
    e = jax.nn.silu(emb_ref[...])
    e = jnp.dot(e, embW_ref[...], preferred_element_type=jnp.float32)
    e = e + embb_ref[...]
    scale = e[:, :D]
    shift = e[:, D:]
    hh = _layernorm(u, sg_ref[0, :], sb_ref[0, :]) * (1.0 + scale) + shift
    hh = jax.nn.silu(hh)
    o = jnp.dot(hh, outW_ref[...], preferred_element_type=jnp.float32)
    out_ref[...] = x_ref[...] + o + outb_ref[...]


def _combine(x2d, moe0, moe1, gate, emb, embW, embb, sn_g, sn_b, outW, outb):
    tdim = embW.shape[0]
    return pl.pallas_call(
        _combine_body,
        grid=(T // TB,),
        in_specs=[
            pl.BlockSpec((TB, D), lambda i: (i, 0)),
            pl.BlockSpec((TB, D), lambda i: (i, 0)),
            pl.BlockSpec((TB, D), lambda i: (i, 0)),
            pl.BlockSpec((NB, 1, TB), lambda i: (0, 0, i)),
            pl.BlockSpec((1, tdim), lambda i: (0, 0)),
            pl.BlockSpec((tdim, 2 * D), lambda i: (0, 0)),
            pl.BlockSpec((1, 2 * D), lambda i: (0, 0)),
            pl.BlockSpec((1, D), lambda i: (0, 0)),
            pl.BlockSpec((1, D), lambda i: (0, 0)),
            pl.BlockSpec((D, D), lambda i: (0, 0)),
            pl.BlockSpec((1, D), lambda i: (0, 0)),
        ],
        out_specs=pl.BlockSpec((TB, D), lambda i: (i, 0)),
        out_shape=jax.ShapeDtypeStruct((T, D), jnp.float32),
    )(x2d, moe0, moe1, gate, emb, embW, embb.reshape(1, 2 * D),
      sn_g.reshape(1, D), sn_b.reshape(1, D), outW, outb.reshape(1, D))


def kernel(x, emb, ln_g, ln_b, gateW, gateb, W1, b1, W2, b2,
           embW, embb, sn_g, sn_b, outW, outb):
    x2d = x.reshape(T, D)

    gate, idxs, cumtok, counts = _routing(x2d, ln_g, ln_b, gateW, gateb)
    perm, dest, be = _finalize(counts, idxs, cumtok)

    sx0 = _sc_gather(x2d, perm[0, 0])
    sx1 = _sc_gather(x2d, perm[1, 0])
    so0 = _ffn(be[0, 0], sx0, ln_g[0], ln_b[0], W1[0], b1[0], W2[0], b2[0])
    so1 = _ffn(be[1, 0], sx1, ln_g[1], ln_b[1], W1[1], b1[1], W2[1], b2[1])
    moe0 = _sc_gather(so0.reshape(PAD, D), dest[0, 0])
    moe1 = _sc_gather(so1.reshape(PAD, D), dest[1, 0])

    out = _combine(x2d, moe0, moe1, gate, emb, embW, embb,
                   sn_g, sn_b, outW, outb)
    return out.reshape(1, T, D)


# speedup vs baseline: 1.9111x; 1.7076x over previous
import functools

import jax
import jax.numpy as jnp
from jax import lax
from jax.experimental import pallas as pl
from jax.experimental.pallas import tpu as pltpu
from jax.experimental.pallas import tpu_sc as plsc

NB = 2
E = 8
D = 768
F = 1024
T = 2048
TB = 256
BM = 128
NBLK = T // BM + E
PAD = NBLK * BM

_SC_WORKERS = 32


def _norm(xb, eps=1e-5):
    m = jnp.mean(xb, axis=-1, keepdims=True)
    v = jnp.mean((xb - m) ** 2, axis=-1, keepdims=True)
    return (xb - m) / jnp.sqrt(v + eps)


def _routing_body(x_ref, g_ref, b_ref, gw_ref, gb_ref,
                  gate_ref, idx_ref, cumtok_ref, counts_ref, carry_ref):
    i = pl.program_id(0)

    @pl.when(i == 0)
    def _():
        carry_ref[...] = jnp.zeros((NB, E), jnp.float32)

    xb = x_ref[0]
    z = _norm(xb)
    r_i = lax.broadcasted_iota(jnp.int32, (TB, TB), 0)
    c_i = lax.broadcasted_iota(jnp.int32, (TB, TB), 1)
    lt = (c_i < r_i).astype(jnp.float32)
    iota_e = lax.broadcasted_iota(jnp.int32, (TB, E), 1)

    for b in range(NB):
        h = z * g_ref[b, 0, :] + b_ref[b, 0, :]
        logits = jnp.dot(h, gw_ref[b], preferred_element_type=jnp.float32)
        logits = logits + gb_ref[b, 0, :]
        lmax = jnp.max(logits, axis=-1, keepdims=True)
        p = jnp.exp(logits - lmax)
        probs = p / jnp.sum(p, axis=-1, keepdims=True)
        gate = jnp.max(probs, axis=-1)
        is_max = probs == gate[:, None]
        idxv = jnp.min(jnp.where(is_max, iota_e, E), axis=-1)
        onehot = (iota_e == idxv[:, None]).astype(jnp.float32)

        cumw = jnp.dot(lt, onehot, preferred_element_type=jnp.float32)
        carry = carry_ref[b, :][None, :]
        cum_excl = cumw + carry
        cumtok = jnp.sum(onehot * cum_excl, axis=-1)

        gate_ref[b, 0, :] = gate
        idx_ref[b, 0, :] = idxv
        cumtok_ref[b, 0, :] = cumtok.astype(jnp.int32)
        new_carry = carry[0, :] + jnp.sum(onehot, axis=0)
        carry_ref[b, :] = new_carry
        counts_ref[b, 0, :] = new_carry.astype(jnp.int32)


def _routing(x, ln_g, ln_b, gateW, gateb):
    return pl.pallas_call(
        _routing_body,
        grid=(T // TB,),
        in_specs=[
            pl.BlockSpec((1, TB, D), lambda i: (0, i, 0)),
            pl.BlockSpec((NB, 1, D), lambda i: (0, 0, 0)),
            pl.BlockSpec((NB, 1, D), lambda i: (0, 0, 0)),
            pl.BlockSpec((NB, D, E), lambda i: (0, 0, 0)),
            pl.BlockSpec((NB, 1, E), lambda i: (0, 0, 0)),
        ],
        out_specs=[
            pl.BlockSpec((NB, 1, TB), lambda i: (0, 0, i)),
            pl.BlockSpec((NB, 1, TB), lambda i: (0, 0, i)),
            pl.BlockSpec((NB, 1, TB), lambda i: (0, 0, i)),
            pl.BlockSpec((NB, 1, E), lambda i: (0, 0, 0)),
        ],
        out_shape=[
            jax.ShapeDtypeStruct((NB, 1, T), jnp.float32),
            jax.ShapeDtypeStruct((NB, 1, T), jnp.int32),
            jax.ShapeDtypeStruct((NB, 1, T), jnp.int32),
            jax.ShapeDtypeStruct((NB, 1, E), jnp.int32),
        ],
        scratch_shapes=[pltpu.VMEM((NB, E), jnp.float32)],
    )(x, ln_g.reshape(NB, 1, D), ln_b.reshape(NB, 1, D), gateW,
      gateb.reshape(NB, 1, E))


def _finalize_body(counts_ref, idx_ref, cumtok_ref, dest_ref, be_ref, ub_ref):
    b = pl.program_id(0)

    counts = counts_ref[0, 0, :]
    nblk_e = (counts + (BM - 1)) // BM
    pc = nblk_e * BM
    e_r = lax.broadcasted_iota(jnp.int32, (E, E), 0)
    e_c = lax.broadcasted_iota(jnp.int32, (E, E), 1)
    lt8 = (e_c < e_r).astype(jnp.float32)
    off = jnp.sum(lt8 * pc.astype(jnp.float32)[None, :], axis=-1)
    ub = jnp.sum(nblk_e)
    iota8 = lax.broadcasted_iota(jnp.int32, (1, E), 1)[0]
    last_e = jnp.max(jnp.where(counts > 0, iota8, 0))

    idxv = idx_ref[0, 0, :]
    cumtok = cumtok_ref[0, 0, :]
    iota_te = lax.broadcasted_iota(jnp.int32, (T, E), 1)
    oh = (iota_te == idxv[:, None]).astype(jnp.float32)
    off_tok = jnp.sum(oh * off[None, :], axis=-1)
    dest_ref[0, 0, :] = cumtok + off_tok.astype(jnp.int32) + b * PAD

    jb = lax.broadcasted_iota(jnp.int32, (NBLK, E), 0) * BM
    off_be = jnp.broadcast_to(off[None, :], (NBLK, E))
    be = jnp.sum((off_be <= jb.astype(jnp.float32)).astype(jnp.int32),
                 axis=-1) - 1
    jblk = lax.broadcasted_iota(jnp.int32, (1, NBLK), 1)[0]
    be = jnp.where(jblk < ub, jnp.clip(be, 0, E - 1), last_e)
    be_ref[0, 0, :] = be
    ub_ref[0, 0, :] = jnp.broadcast_to(ub, (NBLK,))


def _finalize(counts, idxs, cumtok):
    return pl.pallas_call(
        _finalize_body,
        grid=(NB,),
        in_specs=[
            pl.BlockSpec((1, 1, E), lambda b: (b, 0, 0)),
            pl.BlockSpec((1, 1, T), lambda b: (b, 0, 0)),
            pl.BlockSpec((1, 1, T), lambda b: (b, 0, 0)),
        ],
        out_specs=[
            pl.BlockSpec((1, 1, T), lambda b: (b, 0, 0)),
            pl.BlockSpec((1, 1, NBLK), lambda b: (b, 0, 0)),
            pl.BlockSpec((1, 1, NBLK), lambda b: (b, 0, 0)),
        ],
        out_shape=[
            jax.ShapeDtypeStruct((NB, 1, T), jnp.int32),
            jax.ShapeDtypeStruct((NB, 1, NBLK), jnp.int32),
            jax.ShapeDtypeStruct((NB, 1, NBLK), jnp.int32),
        ],
    )(counts, idxs, cumtok)


def _sc_scatter(x2d, dest_flat):
    per_w = (NB * T) // _SC_WORKERS
    mesh = plsc.VectorSubcoreMesh(core_axis_name="c", subcore_axis_name="s")

    @functools.partial(
        pl.kernel,
        mesh=mesh,
        out_type=jax.ShapeDtypeStruct((NB * PAD, D), jnp.float32),
        scratch_types=[
            pltpu.VMEM((per_w,), jnp.int32),
            pltpu.VMEM((per_w, D), jnp.float32),
            pltpu.SemaphoreType.DMA,
        ],
    )
    def k(x_hbm, dest_hbm, out_hbm, idx_v, rows_v, sem):
        wid = lax.axis_index("s") * 2 + lax.axis_index("c")
        base = wid * per_w
        off_x = lax.rem(base, T)
        pltpu.sync_copy(dest_hbm.at[pl.ds(base, per_w)], idx_v)
        pltpu.sync_copy(x_hbm.at[pl.ds(off_x, per_w)], rows_v)
        pltpu.async_copy(rows_v, out_hbm.at[idx_v], sem).wait()

    return k(x2d, dest_flat)


def _sc_gather(table, dest_flat):
    per_w = (NB * T) // _SC_WORKERS
    mesh = plsc.VectorSubcoreMesh(core_axis_name="c", subcore_axis_name="s")

    @functools.partial(
        pl.kernel,
        mesh=mesh,
        out_type=jax.ShapeDtypeStruct((NB * T, D), jnp.float32),
        scratch_types=[
            pltpu.VMEM((per_w,), jnp.int32),
            pltpu.VMEM((per_w, D), jnp.float32),
            pltpu.SemaphoreType.DMA,
        ],
    )
    def k(table_hbm, dest_hbm, out_hbm, idx_v, rows_v, sem):
        wid = lax.axis_index("s") * 2 + lax.axis_index("c")
        base = wid * per_w
        pltpu.sync_copy(dest_hbm.at[pl.ds(base, per_w)], idx_v)
        pltpu.async_copy(table_hbm.at[idx_v], rows_v, sem).wait()
        pltpu.sync_copy(rows_v, out_hbm.at[pl.ds(base, per_w)])

    return k(table, dest_flat)


def _ffn_body(be_ref, ub_ref, x_ref, g_ref, b_ref, w1_ref, b1_ref, w2_ref,
              b2_ref, out_ref):
    b = pl.program_id(0)
    j = pl.program_id(1)

    @pl.when(j < ub_ref[b, 0, 0])
    def _():
        xb = x_ref[0]
        m = jnp.mean(xb, axis=-1, keepdims=True)
        v = jnp.mean((xb - m) ** 2, axis=-1, keepdims=True)
        h = (xb - m) / jnp.sqrt(v + 1e-5) * g_ref[0, 0, :] + b_ref[0, 0, :]
        h1 = jnp.dot(h, w1_ref[0, 0], preferred_element_type=jnp.float32)
        h1 = jax.nn.gelu(h1 + b1_ref[0, 0, 0])
        h2 = jnp.dot(h1, w2_ref[0, 0], preferred_element_type=jnp.float32)
        out_ref[0] = h2 + b2_ref[0, 0, 0]


def _ffn(be, ub, sorted_x, ln_g, ln_b, W1, b1, W2, b2):
    grid_spec = pltpu.PrefetchScalarGridSpec(
        num_scalar_prefetch=2,
        grid=(NB, NBLK),
        in_specs=[
            pl.BlockSpec((1, BM, D), lambda b, j, s, u: (b, j, 0)),
            pl.BlockSpec((1, 1, D), lambda b, j, s, u: (b, 0, 0)),
            pl.BlockSpec((1, 1, D), lambda b, j, s, u: (b, 0, 0)),
            pl.BlockSpec((1, 1, D, F), lambda b, j, s, u: (b, s[b, 0, j], 0, 0)),
            pl.BlockSpec((1, 1, 1, F), lambda b, j, s, u: (b, s[b, 0, j], 0, 0)),
            pl.BlockSpec((1, 1, F, D), lambda b, j, s, u: (b, s[b, 0, j], 0, 0)),
            pl.BlockSpec((1, 1, 1, D), lambda b, j, s, u: (b, s[b, 0, j], 0, 0)),
        ],
        out_specs=pl.BlockSpec((1, BM, D), lambda b, j, s, u: (b, j, 0)),
    )
    return pl.pallas_call(
        _ffn_body,
        grid_spec=grid_spec,
        out_shape=jax.ShapeDtypeStruct((NB, PAD, D), jnp.float32),
    )(be, ub, sorted_x, ln_g.reshape(NB, 1, D), ln_b.reshape(NB, 1, D),
      W1, b1.reshape(NB, E, 1, F), W2, b2.reshape(NB, E, 1, D))


def _combine_body(x_ref, moe_ref, gate_ref, emb_ref, embW_ref, embb_ref,
                  sg_ref, sb_ref, outW_ref, outb_ref, out_ref):
    g0 = gate_ref[0, 0, :]
    g1 = gate_ref[1, 0, :]
    u = (moe_ref[0] * g0[:, None] + moe_ref[1] * g1[:, None]) * 0.5
    e = jax.nn.silu(emb_ref[...])
    e = jnp.dot(e, embW_ref[...], preferred_element_type=jnp.float32)
    e = e + embb_ref[...]
    scale = e[:, :D]
    shift = e[:, D:]
    m = jnp.mean(u, axis=-1, keepdims=True)
    v = jnp.mean((u - m) ** 2, axis=-1, keepdims=True)
    hh = (u - m) / jnp.sqrt(v + 1e-5) * sg_ref[0, :] + sb_ref[0, :]
    hh = hh * (1.0 + scale) + shift
    hh = jax.nn.silu(hh)
    o = jnp.dot(hh, outW_ref[...], preferred_element_type=jnp.float32)
    out_ref[0] = x_ref[0] + o + outb_ref[...]


def _combine(x, moe, gate, emb, embW, embb, sn_g, sn_b, outW, outb):
    tdim = embW.shape[0]
    return pl.pallas_call(
        _combine_body,
        grid=(T // TB,),
        in_specs=[
            pl.BlockSpec((1, TB, D), lambda i: (0, i, 0)),
            pl.BlockSpec((NB, TB, D), lambda i: (0, i, 0)),
            pl.BlockSpec((NB, 1, TB), lambda i: (0, 0, i)),
            pl.BlockSpec((1, tdim), lambda i: (0, 0)),
            pl.BlockSpec((tdim, 2 * D), lambda i: (0, 0)),
            pl.BlockSpec((1, 2 * D), lambda i: (0, 0)),
            pl.BlockSpec((1, D), lambda i: (0, 0)),
            pl.BlockSpec((1, D), lambda i: (0, 0)),
            pl.BlockSpec((D, D), lambda i: (0, 0)),
            pl.BlockSpec((1, D), lambda i: (0, 0)),
        ],
        out_specs=pl.BlockSpec((1, TB, D), lambda i: (0, i, 0)),
        out_shape=jax.ShapeDtypeStruct((1, T, D), jnp.float32),
    )(x, moe, gate, emb, embW, embb.reshape(1, 2 * D),
      sn_g.reshape(1, D), sn_b.reshape(1, D), outW, outb.reshape(1, D))


def kernel(x, emb, ln_g, ln_b, gateW, gateb, W1, b1, W2, b2,
           embW, embb, sn_g, sn_b, outW, outb):
    gate, idxs, cumtok, counts = _routing(x, ln_g, ln_b, gateW, gateb)
    dest, be, ub = _finalize(counts, idxs, cumtok)
    dest_flat = dest.reshape(NB * T)

    sorted_x = _sc_scatter(x.reshape(T, D), dest_flat)
    sorted_out = _ffn(be, ub, sorted_x.reshape(NB, PAD, D),
                      ln_g, ln_b, W1, b1, W2, b2)
    moe = _sc_gather(sorted_out.reshape(NB * PAD, D), dest_flat)

    out = _combine(x, moe.reshape(NB, T, D), gate, emb, embW, embb,
                   sn_g, sn_b, outW, outb)
    return out


# speedup vs baseline: 1.9588x; 1.0249x over previous
import functools

import jax
import jax.numpy as jnp
from jax import lax
from jax.experimental import pallas as pl
from jax.experimental.pallas import tpu as pltpu
from jax.experimental.pallas import tpu_sc as plsc

NB = 2
E = 8
D = 768
F = 1024
T = 2048
TB = 256
BM = 128
NBLK = T // BM + E
PAD = NBLK * BM

_SC_WORKERS = 32


def _norm(xb, eps=1e-5):
    m = jnp.mean(xb, axis=-1, keepdims=True)
    v = jnp.mean((xb - m) ** 2, axis=-1, keepdims=True)
    return (xb - m) / jnp.sqrt(v + eps)


def _routing_body(x_ref, g_ref, b_ref, gw_ref, gb_ref,
                  gate_ref, idx_ref, cumtok_ref, counts_ref, carry_ref):
    i = pl.program_id(0)

    @pl.when(i == 0)
    def _():
        carry_ref[...] = jnp.zeros((NB, E), jnp.float32)

    xb = x_ref[0]
    z = _norm(xb)
    r_i = lax.broadcasted_iota(jnp.int32, (TB, TB), 0)
    c_i = lax.broadcasted_iota(jnp.int32, (TB, TB), 1)
    lt = (c_i < r_i).astype(jnp.float32)
    iota_e = lax.broadcasted_iota(jnp.int32, (TB, E), 1)

    for b in range(NB):
        h = z * g_ref[b, 0, :] + b_ref[b, 0, :]
        logits = jnp.dot(h, gw_ref[b], preferred_element_type=jnp.float32)
        logits = logits + gb_ref[b, 0, :]
        lmax = jnp.max(logits, axis=-1, keepdims=True)
        p = jnp.exp(logits - lmax)
        probs = p / jnp.sum(p, axis=-1, keepdims=True)
        gate = jnp.max(probs, axis=-1)
        is_max = probs == gate[:, None]
        idxv = jnp.min(jnp.where(is_max, iota_e, E), axis=-1)
        onehot = (iota_e == idxv[:, None]).astype(jnp.float32)

        cumw = jnp.dot(lt, onehot, preferred_element_type=jnp.float32)
        carry = carry_ref[b, :][None, :]
        cum_excl = cumw + carry
        cumtok = jnp.sum(onehot * cum_excl, axis=-1)

        gate_ref[b, 0, :] = gate
        idx_ref[b, 0, :] = idxv
        cumtok_ref[b, 0, :] = cumtok.astype(jnp.int32)
        new_carry = carry[0, :] + jnp.sum(onehot, axis=0)
        carry_ref[b, :] = new_carry
        counts_ref[b, 0, :] = new_carry.astype(jnp.int32)


def _routing(x, ln_g, ln_b, gateW, gateb):
    return pl.pallas_call(
        _routing_body,
        grid=(T // TB,),
        in_specs=[
            pl.BlockSpec((1, TB, D), lambda i: (0, i, 0)),
            pl.BlockSpec((NB, 1, D), lambda i: (0, 0, 0)),
            pl.BlockSpec((NB, 1, D), lambda i: (0, 0, 0)),
            pl.BlockSpec((NB, D, E), lambda i: (0, 0, 0)),
            pl.BlockSpec((NB, 1, E), lambda i: (0, 0, 0)),
        ],
        out_specs=[
            pl.BlockSpec((NB, 1, TB), lambda i: (0, 0, i)),
            pl.BlockSpec((NB, 1, TB), lambda i: (0, 0, i)),
            pl.BlockSpec((NB, 1, TB), lambda i: (0, 0, i)),
            pl.BlockSpec((NB, 1, E), lambda i: (0, 0, 0)),
        ],
        out_shape=[
            jax.ShapeDtypeStruct((NB, 1, T), jnp.float32),
            jax.ShapeDtypeStruct((NB, 1, T), jnp.int32),
            jax.ShapeDtypeStruct((NB, 1, T), jnp.int32),
            jax.ShapeDtypeStruct((NB, 1, E), jnp.int32),
        ],
        scratch_shapes=[pltpu.VMEM((NB, E), jnp.float32)],
    )(x, ln_g.reshape(NB, 1, D), ln_b.reshape(NB, 1, D), gateW,
      gateb.reshape(NB, 1, E))


def _finalize_body(counts_ref, idx_ref, cumtok_ref, dest_ref, be_ref, ub_ref):
    b = pl.program_id(0)

    counts = counts_ref[0, 0, :]
    nblk_e = (counts + (BM - 1)) // BM
    pc = nblk_e * BM
    e_r = lax.broadcasted_iota(jnp.int32, (E, E), 0)
    e_c = lax.broadcasted_iota(jnp.int32, (E, E), 1)
    lt8 = (e_c < e_r).astype(jnp.float32)
    off = jnp.sum(lt8 * pc.astype(jnp.float32)[None, :], axis=-1)
    ub = jnp.sum(nblk_e)
    iota8 = lax.broadcasted_iota(jnp.int32, (1, E), 1)[0]
    last_e = jnp.max(jnp.where(counts > 0, iota8, 0))

    idxv = idx_ref[0, 0, :]
    cumtok = cumtok_ref[0, 0, :]
    iota_te = lax.broadcasted_iota(jnp.int32, (T, E), 1)
    oh = (iota_te == idxv[:, None]).astype(jnp.float32)
    off_tok = jnp.sum(oh * off[None, :], axis=-1)
    dest_ref[0, 0, :] = cumtok + off_tok.astype(jnp.int32)

    jb = lax.broadcasted_iota(jnp.int32, (NBLK, E), 0) * BM
    off_be = jnp.broadcast_to(off[None, :], (NBLK, E))
    be = jnp.sum((off_be <= jb.astype(jnp.float32)).astype(jnp.int32),
                 axis=-1) - 1
    jblk = lax.broadcasted_iota(jnp.int32, (1, NBLK), 1)[0]
    be = jnp.where(jblk < ub, jnp.clip(be, 0, E - 1), last_e)
    be_ref[0, 0, :] = be
    ub_ref[0, 0, :] = jnp.broadcast_to(ub, (NBLK,))


def _finalize(counts, idxs, cumtok):
    return pl.pallas_call(
        _finalize_body,
        grid=(NB,),
        in_specs=[
            pl.BlockSpec((1, 1, E), lambda b: (b, 0, 0)),
            pl.BlockSpec((1, 1, T), lambda b: (b, 0, 0)),
            pl.BlockSpec((1, 1, T), lambda b: (b, 0, 0)),
        ],
        out_specs=[
            pl.BlockSpec((1, 1, T), lambda b: (b, 0, 0)),
            pl.BlockSpec((1, 1, NBLK), lambda b: (b, 0, 0)),
            pl.BlockSpec((1, 1, NBLK), lambda b: (b, 0, 0)),
        ],
        out_shape=[
            jax.ShapeDtypeStruct((NB, 1, T), jnp.int32),
            jax.ShapeDtypeStruct((NB, 1, NBLK), jnp.int32),
            jax.ShapeDtypeStruct((NB, 1, NBLK), jnp.int32),
        ],
    )(counts, idxs, cumtok)


def _sc_scatter(x2d, dest_b):
    per_w = T // _SC_WORKERS
    mesh = plsc.VectorSubcoreMesh(core_axis_name="c", subcore_axis_name="s")

    @functools.partial(
        pl.kernel,
        mesh=mesh,
        out_type=jax.ShapeDtypeStruct((PAD, D), jnp.float32),
        scratch_types=[
            pltpu.VMEM((per_w,), jnp.int32),
            pltpu.VMEM((per_w, D), jnp.float32),
            pltpu.SemaphoreType.DMA,
        ],
    )
    def k(x_hbm, dest_hbm, out_hbm, idx_v, rows_v, sem):
        wid = lax.axis_index("s") * 2 + lax.axis_index("c")
        base = wid * per_w
        pltpu.sync_copy(dest_hbm.at[pl.ds(base, per_w)], idx_v)
        pltpu.sync_copy(x_hbm.at[pl.ds(base, per_w)], rows_v)
        pltpu.async_copy(rows_v, out_hbm.at[idx_v], sem).wait()

    return k(x2d, dest_b)


def _sc_gather(table, dest_b):
    per_w = T // _SC_WORKERS
    mesh = plsc.VectorSubcoreMesh(core_axis_name="c", subcore_axis_name="s")

    @functools.partial(
        pl.kernel,
        mesh=mesh,
        out_type=jax.ShapeDtypeStruct((T, D), jnp.float32),
        scratch_types=[
            pltpu.VMEM((per_w,), jnp.int32),
            pltpu.VMEM((per_w, D), jnp.float32),
            pltpu.SemaphoreType.DMA,
        ],
    )
    def k(table_hbm, dest_hbm, out_hbm, idx_v, rows_v, sem):
        wid = lax.axis_index("s") * 2 + lax.axis_index("c")
        base = wid * per_w
        pltpu.sync_copy(dest_hbm.at[pl.ds(base, per_w)], idx_v)
        pltpu.async_copy(table_hbm.at[idx_v], rows_v, sem).wait()
        pltpu.sync_copy(rows_v, out_hbm.at[pl.ds(base, per_w)])

    return k(table, dest_b)


def _ffn(b, be, ub, sorted_x, ln_g, ln_b, W1, b1, W2, b2):
    def body(be_ref, ub_ref, x_ref, g_ref, b_ref, w1_ref, b1_ref, w2_ref,
             b2_ref, out_ref):
        j = pl.program_id(0)

        @pl.when(j < ub_ref[b, 0, 0])
        def _():
            xb = x_ref[...]
            m = jnp.mean(xb, axis=-1, keepdims=True)
            v = jnp.mean((xb - m) ** 2, axis=-1, keepdims=True)
            h = ((xb - m) / jnp.sqrt(v + 1e-5) * g_ref[0, 0, :]
                 + b_ref[0, 0, :])
            h1 = jnp.dot(h, w1_ref[0, 0], preferred_element_type=jnp.float32)
            h1 = jax.nn.gelu(h1 + b1_ref[0, 0, 0])
            h2 = jnp.dot(h1, w2_ref[0, 0], preferred_element_type=jnp.float32)
            out_ref[...] = h2 + b2_ref[0, 0, 0]

    def jclamp(j, u):
        return jnp.minimum(j, u[b, 0, 0] - 1)

    grid_spec = pltpu.PrefetchScalarGridSpec(
        num_scalar_prefetch=2,
        grid=(NBLK,),
        in_specs=[
            pl.BlockSpec((BM, D), lambda j, s, u: (jclamp(j, u), 0)),
            pl.BlockSpec((1, 1, D), lambda j, s, u: (b, 0, 0)),
            pl.BlockSpec((1, 1, D), lambda j, s, u: (b, 0, 0)),
            pl.BlockSpec((1, 1, D, F), lambda j, s, u: (b, s[b, 0, j], 0, 0)),
            pl.BlockSpec((1, 1, 1, F), lambda j, s, u: (b, s[b, 0, j], 0, 0)),
            pl.BlockSpec((1, 1, F, D), lambda j, s, u: (b, s[b, 0, j], 0, 0)),
            pl.BlockSpec((1, 1, 1, D), lambda j, s, u: (b, s[b, 0, j], 0, 0)),
        ],
        out_specs=pl.BlockSpec((BM, D), lambda j, s, u: (jclamp(j, u), 0)),
    )
    return pl.pallas_call(
        body,
        grid_spec=grid_spec,
        out_shape=jax.ShapeDtypeStruct((PAD, D), jnp.float32),
    )(be, ub, sorted_x, ln_g.reshape(NB, 1, D), ln_b.reshape(NB, 1, D),
      W1, b1.reshape(NB, E, 1, F), W2, b2.reshape(NB, E, 1, D))


def _combine_body(x_ref, moe0_ref, moe1_ref, gate_ref, emb_ref, embW_ref,
                  embb_ref, sg_ref, sb_ref, outW_ref, outb_ref, out_ref):
    g0 = gate_ref[0, 0, :]
    g1 = gate_ref[1, 0, :]
    u = (moe0_ref[...] * g0[:, None] + moe1_ref[...] * g1[:, None]) * 0.5
    e = jax.nn.silu(emb_ref[...])
    e = jnp.dot(e, embW_ref[...], preferred_element_type=jnp.float32)
    e = e + embb_ref[...]
    scale = e[:, :D]
    shift = e[:, D:]
    m = jnp.mean(u, axis=-1, keepdims=True)
    v = jnp.mean((u - m) ** 2, axis=-1, keepdims=True)
    hh = (u - m) / jnp.sqrt(v + 1e-5) * sg_ref[0, :] + sb_ref[0, :]
    hh = hh * (1.0 + scale) + shift
    hh = jax.nn.silu(hh)
    o = jnp.dot(hh, outW_ref[...], preferred_element_type=jnp.float32)
    out_ref[0] = x_ref[0] + o + outb_ref[...]


def _combine(x, moe0, moe1, gate, emb, embW, embb, sn_g, sn_b, outW, outb):
    tdim = embW.shape[0]
    return pl.pallas_call(
        _combine_body,
        grid=(T // TB,),
        in_specs=[
            pl.BlockSpec((1, TB, D), lambda i: (0, i, 0)),
            pl.BlockSpec((TB, D), lambda i: (i, 0)),
            pl.BlockSpec((TB, D), lambda i: (i, 0)),
            pl.BlockSpec((NB, 1, TB), lambda i: (0, 0, i)),
            pl.BlockSpec((1, tdim), lambda i: (0, 0)),
            pl.BlockSpec((tdim, 2 * D), lambda i: (0, 0)),
            pl.BlockSpec((1, 2 * D), lambda i: (0, 0)),
            pl.BlockSpec((1, D), lambda i: (0, 0)),
            pl.BlockSpec((1, D), lambda i: (0, 0)),
            pl.BlockSpec((D, D), lambda i: (0, 0)),
            pl.BlockSpec((1, D), lambda i: (0, 0)),
        ],
        out_specs=pl.BlockSpec((1, TB, D), lambda i: (0, i, 0)),
        out_shape=jax.ShapeDtypeStruct((1, T, D), jnp.float32),
    )(x, moe0, moe1, gate, emb, embW, embb.reshape(1, 2 * D),
      sn_g.reshape(1, D), sn_b.reshape(1, D), outW, outb.reshape(1, D))


def kernel(x, emb, ln_g, ln_b, gateW, gateb, W1, b1, W2, b2,
           embW, embb, sn_g, sn_b, outW, outb):
    gate, idxs, cumtok, counts = _routing(x, ln_g, ln_b, gateW, gateb)
    dest, be, ub = _finalize(counts, idxs, cumtok)
    x2d = x.reshape(T, D)
    dest0 = dest[0, 0]
    dest1 = dest[1, 0]

    sx0 = _sc_scatter(x2d, dest0)
    sx1 = _sc_scatter(x2d, dest1)
    so0 = _ffn(0, be, ub, sx0, ln_g, ln_b, W1, b1, W2, b2)
    moe0 = _sc_gather(so0, dest0)
    so1 = _ffn(1, be, ub, sx1, ln_g, ln_b, W1, b1, W2, b2)
    moe1 = _sc_gather(so1, dest1)

    out = _combine(x, moe0, moe1, gate, emb, embW, embb,
                   sn_g, sn_b, outW, outb)
    return out
